# baseline (speedup 1.0000x reference)
import functools

import jax
import jax.numpy as jnp
from jax import lax
from jax.experimental import pallas as pl
from jax.experimental.pallas import tpu as pltpu
from jax.experimental.pallas import tpu_sc as plsc

F32 = jnp.float32
HID = 128
NB = 10000
NC = 1000
NN = NB + NC
NUM_LAYERS = 2
K_ED = 64
PIPE_D = 4
NW = 32


def _combine(Wsrc, bsrc, Wdst, bdst, Wmlp, bmlp, eps, l):
    hp = jax.lax.Precision.HIGHEST
    i_bb, i_bc, i_cc, i_cb = l * 4 + 0, l * 4 + 1, l * 4 + 2, l * 4 + 3

    def wm(i, j):
        return jnp.dot(Wsrc[i][...] if j is None else j, Wmlp[i][...],
                       preferred_element_type=F32, precision=hp)

    Mb = (1.0 + eps[0, i_bb]) * wm(i_bb, Wdst[i_bb][...]) \
       + (1.0 + eps[0, i_cb]) * wm(i_cb, Wdst[i_cb][...])
    Mc = (1.0 + eps[0, i_bc]) * wm(i_bc, Wdst[i_bc][...]) \
       + (1.0 + eps[0, i_cc]) * wm(i_cc, Wdst[i_cc][...])
    Wb = jnp.concatenate([wm(i_bb, None), wm(i_bc, None), Mb], axis=1)
    Wc = jnp.concatenate([wm(i_cb, None), wm(i_cc, None), Mc], axis=1)

    def bvec(i):
        return bsrc[i][...].reshape(1, HID)

    cb_const = wm(i_bb, bdst[i_bb][...].reshape(1, HID)) + bmlp[i_bb][...].reshape(1, HID) \
             + wm(i_cb, bdst[i_cb][...].reshape(1, HID)) + bmlp[i_cb][...].reshape(1, HID)
    cc_const = wm(i_bc, bdst[i_bc][...].reshape(1, HID)) + bmlp[i_bc][...].reshape(1, HID) \
             + wm(i_cc, bdst[i_cc][...].reshape(1, HID)) + bmlp[i_cc][...].reshape(1, HID)
    bbias = jnp.concatenate([wm(i_bb, bvec(i_bb)), wm(i_bc, bvec(i_bc)), cb_const], axis=1)
    cbias = jnp.concatenate([wm(i_cb, bvec(i_cb)), wm(i_cc, bvec(i_cc)), cc_const], axis=1)
    return Wb, bbias, Wc, cbias


def _split_writes(yb, yc, Hb_o, Hc_o, sb_o, sc_o):
    Hb_o[0:NB, :] = yb[:, 0:HID]
    Hb_o[NB:NN, :] = yc[:, 0:HID]
    Hc_o[0:NB, :] = yb[:, HID:2 * HID]
    Hc_o[NB:NN, :] = yc[:, HID:2 * HID]
    sb_o[...] = yb[:, 2 * HID:]
    sc_o[...] = yc[:, 2 * HID:]


def _tc1_body(old, xc, Wa, ba, Wsrc, bsrc, Wdst, bdst, Wmlp, bmlp, eps,
              xb_o, Hb_o, Hc_o, sb_o, sc_o, Wb1_o, bb1_o, Wc1_o, bc1_o):
    Wb0, bb0, Wc0, bc0 = _combine(Wsrc, bsrc, Wdst, bdst, Wmlp, bmlp, eps, 0)
    Wb1, bb1, Wc1, bc1 = _combine(Wsrc, bsrc, Wdst, bdst, Wmlp, bmlp, eps, 1)
    Wb1_o[...] = Wb1
    bb1_o[...] = bb1
    Wc1_o[...] = Wc1
    bc1_o[...] = bc1
    xb = jnp.dot(old[...], Wa[...], preferred_element_type=F32) + ba[...]
    xb_o[...] = xb
    yb = jnp.dot(xb, Wb0, preferred_element_type=F32) + bb0
    yc = jnp.dot(xc[...], Wc0, preferred_element_type=F32) + bc0
    _split_writes(yb, yc, Hb_o, Hc_o, sb_o, sc_o)


def _tc1_call(old, xc, Wa, ba, Wsrc, bsrc, Wdst, bdst, Wmlp, bmlp, eps):
    b = jax.ShapeDtypeStruct((NB, HID), F32)
    h = jax.ShapeDtypeStruct((NN, HID), F32)
    cs = jax.ShapeDtypeStruct((NC, HID), F32)
    w = jax.ShapeDtypeStruct((HID, 3 * HID), F32)
    bi = jax.ShapeDtypeStruct((1, 3 * HID), F32)
    return pl.pallas_call(
        _tc1_body,
        out_shape=(b, h, h, b, cs, w, bi, w, bi),
    )(old, xc, Wa, ba, Wsrc, bsrc, Wdst, bdst, Wmlp, bmlp, eps)


def _tc2_body(xb, ab0, ab1, sb, xc, ac0, ac1, sc, Wb, bbias, Wc, cbias,
              xb_o, Hb_o, Hc_o, sb_o, xc_o, sc_o):
    xb1 = xb[...] + jax.nn.gelu(ab0[...] + ab1[...] + sb[...])
    xb_o[...] = xb1
    xc1 = xc[...] + jax.nn.gelu(ac0[...] + ac1[...] + sc[...])
    xc_o[...] = xc1
    yb = jnp.dot(xb1, Wb[...], preferred_element_type=F32) + bbias[...]
    yc = jnp.dot(xc1, Wc[...], preferred_element_type=F32) + cbias[...]
    _split_writes(yb, yc, Hb_o, Hc_o, sb_o, sc_o)


def _tc2_call(xb, ab0, ab1, sb, xc, ac0, ac1, sc, Wb, bbias, Wc, cbias):
    b = jax.ShapeDtypeStruct((NB, HID), F32)
    h = jax.ShapeDtypeStruct((NN, HID), F32)
    cs = jax.ShapeDtypeStruct((NC, HID), F32)
    return pl.pallas_call(
        _tc2_body,
        out_shape=(b, h, h, b, cs, cs),
    )(xb, ab0, ab1, sb, xc, ac0, ac1, sc, Wb, bbias, Wc, cbias)


def _tc3_body(xb, ab0, ab1, sb, xc, ac0, ac1, sc, xb_o, xc_o):
    xb_o[...] = xb[...] + jax.nn.gelu(ab0[...] + ab1[...] + sb[...])
    xc_o[...] = xc[...] + jax.nn.gelu(ac0[...] + ac1[...] + sc[...])


def _tc3_call(xb, ab0, ab1, sb, xc, ac0, ac1, sc):
    return pl.pallas_call(
        _tc3_body,
        out_shape=(jax.ShapeDtypeStruct((NB, HID), F32),
                   jax.ShapeDtypeStruct((NC, HID), F32)),
    )(xb, ab0, ab1, sb, xc, ac0, ac1, sc)


def _sc_body(Hb, Hc, pk_b, pw_b, pk_c, pw_c, zeros,
             aggb_o, aggc_o,
             aggb_sh, aggc_sh,
             ib00, ib01, ib10, ib11, ib20, ib21, ib30, ib31,
             wf00, wf01, wf10, wf11, wf20, wf21, wf30, wf31,
             rw0, rw1, rw2, rw3,
             is00, is01, is10, is11, is20, is21, is30, is31,
             gs0, gs1, gs2, gs3):
    ibufs = [[ib00, ib01], [ib10, ib11], [ib20, ib21], [ib30, ib31]]
    wfs = [[wf00, wf01], [wf10, wf11], [wf20, wf21], [wf30, wf31]]
    rows = [rw0, rw1, rw2, rw3]
    isems = [[is00, is01], [is10, is11], [is20, is21], [is30, is31]]
    gsems = [gs0, gs1, gs2, gs3]
    c = lax.axis_index("c")
    s = lax.axis_index("s")
    wid = c * 16 + s

    @pl.when(s < 10)
    def _():
        pltpu.sync_copy(zeros.at[:], aggb_sh.at[pl.ds(s * 1000, 1000)])

    @pl.when(s < 5)
    def _():
        pltpu.sync_copy(zeros.at[pl.ds(0, 200)], aggc_sh.at[pl.ds(s * 200, 200)])

    plsc.subcore_barrier()

    def do_rel(htab, pk4, pw4, agg_sh):
        chunks = pk4.shape[1]
        D = PIPE_D
        assert chunks % (2 * D) == 0 and chunks >= 2 * D

        def idx_start(g, d, p):
            pltpu.async_copy(pk4.at[wid, g], ibufs[d][p], isems[d][p])
            pltpu.async_copy(pw4.at[wid, g], wfs[d][p].at[:, pl.ds(0, K_ED)],
                             isems[d][p])

        def idx_wait(d, p):
            pltpu.make_async_copy(pk4.at[wid, 0], ibufs[d][p], isems[d][p]).wait()
            pltpu.make_async_copy(pw4.at[wid, 0], wfs[d][p].at[:, pl.ds(0, K_ED)],
                                  isems[d][p]).wait()

        def gather_start(d, p):
            pltpu.async_copy(htab.at[ibufs[d][p].at[0]], rows[d], gsems[d])

        def gather_wait(d, p):
            pltpu.make_async_copy(htab.at[ibufs[d][p].at[0]], rows[d], gsems[d]).wait()

        def process(d, p):
            ib, wf, rw = ibufs[d][p], wfs[d][p], rows[d]

            def scale4(t4, c2):
                base = t4 * 4
                wv = wf[0, pl.ds(base, 16)]
                for u in range(4):
                    e = base + u
                    w = wv[u]
                    for j in range(8):
                        sl = pl.ds(j * 16, 16)
                        rw[e, sl] = rw[e, sl] * w
                return c2

            lax.fori_loop(0, K_ED // 4, scale4, 0)
            pltpu.sync_copy(rw, agg_sh.at[ib.at[1]], add=True)

        for g in range(2 * D):
            idx_start(g, g % D, (g // D) % 2)
        for g in range(D):
            idx_wait(g % D, 0)
            gather_start(g % D, 0)

        def body2D(t2, carry):
            for half in range(2):
                for d in range(D):
                    g = (t2 * 2 + half) * D + d
                    p = half
                    gather_wait(d, p)
                    process(d, p)

                    @pl.when(g + 2 * D < chunks)
                    def _(g=g, d=d, p=p):
                        idx_start(g + 2 * D, d, p)

                    @pl.when(g + D < chunks)
                    def _(d=d, p=p):
                        idx_wait(d, 1 - p)
                        gather_start(d, 1 - p)
            return carry

        lax.fori_loop(0, chunks // (2 * D), body2D, 0)

    do_rel(Hb, pk_b, pw_b, aggb_sh)
    do_rel(Hc, pk_c, pw_c, aggc_sh)

    plsc.subcore_barrier()

    @pl.when(s < 10)
    def _():
        pltpu.sync_copy(aggb_sh.at[pl.ds(s * 1000, 1000)],
                        aggb_o.at[c, pl.ds(s * 1000, 1000)])

    @pl.when(s < 5)
    def _():
        pltpu.sync_copy(aggc_sh.at[pl.ds(s * 200, 200)],
                        aggc_o.at[c, pl.ds(s * 200, 200)])


@functools.cache
def _get_sc_call():
  return functools.partial(
    pl.kernel,
    mesh=plsc.VectorSubcoreMesh(core_axis_name="c", subcore_axis_name="s",
                                num_cores=2, num_subcores=16),
    compiler_params=pltpu.CompilerParams(use_tc_tiling_on_sc=False),
    out_type=(jax.ShapeDtypeStruct((2, NB, HID), F32),
              jax.ShapeDtypeStruct((2, NC, HID), F32)),
    scratch_types=[
        pltpu.VMEM_SHARED((NB, HID), F32),
        pltpu.VMEM_SHARED((NC, HID), F32),
        *([pltpu.VMEM((2, K_ED), jnp.int32)] * 8),
        *([pltpu.VMEM((1, K_ED + 16), F32)] * 8),
        *([pltpu.VMEM((K_ED, HID), F32)] * 4),
        *([pltpu.SemaphoreType.DMA] * 12),
    ],
  )(_sc_body)


def _pack_edges(src, dst, w):
    per = src.shape[0] // NW
    chunks = -(-per // K_ED)
    chunks = -(-chunks // (2 * PIPE_D)) * (2 * PIPE_D)
    pad = chunks * K_ED - per

    def p2(x):
        return jnp.pad(x.reshape(NW, per), ((0, 0), (0, pad))).reshape(
            NW, chunks, 1, K_ED)

    return jnp.concatenate([p2(src), p2(dst)], axis=2), p2(w)


def kernel(old_data, x_base, x_centroid, edge_index_b2b, edge_index_b2c,
           edge_index_c2c, edge_index_c2b, edge_weight_b2b, edge_weight_b2c,
           edge_weight_c2c, edge_weight_c2b, batch_base, batch_centroid,
           has_edge_attr, params):
    p = params
    ets = [(l, et) for l in range(NUM_LAYERS) for et in ("bb", "bc", "cc", "cb")]
    Wsrc = jnp.stack([p[f"{l}_{et}"]["W_src"] for l, et in ets])
    bsrc = jnp.stack([p[f"{l}_{et}"]["b_src"] for l, et in ets])
    Wdst = jnp.stack([p[f"{l}_{et}"]["W_dst"] for l, et in ets])
    bdst = jnp.stack([p[f"{l}_{et}"]["b_dst"] for l, et in ets])
    Wmlp = jnp.stack([p[f"{l}_{et}"]["W_mlp"] for l, et in ets])
    bmlp = jnp.stack([p[f"{l}_{et}"]["b_mlp"] for l, et in ets])
    eps = jnp.stack([p[f"{l}_{et}"]["eps"] for l, et in ets]).reshape(1, 8)

    src_b = jnp.concatenate([edge_index_b2b[0], edge_index_c2b[0] + NB])
    dst_b = jnp.concatenate([edge_index_b2b[1], edge_index_c2b[1]])
    w_b = jnp.concatenate([edge_weight_b2b, edge_weight_c2b])
    src_c = jnp.concatenate([edge_index_b2c[0], edge_index_c2c[0] + NB])
    dst_c = jnp.concatenate([edge_index_b2c[1], edge_index_c2c[1]])
    w_c = jnp.concatenate([edge_weight_b2c, edge_weight_c2c])
    pk_b, pw_b = _pack_edges(src_b, dst_b, w_b)
    pk_c, pw_c = _pack_edges(src_c, dst_c, w_c)

    zeros = jnp.zeros((1000, HID), F32)

    (xb, Hb, Hc, sb, scn, Wb1, bb1, Wc1, bc1) = _tc1_call(
        old_data, x_centroid, p["W_atom"], p["b_atom"].reshape(1, HID),
        Wsrc, bsrc, Wdst, bdst, Wmlp, bmlp, eps)

    aggb, aggc = _get_sc_call()(Hb, Hc, pk_b, pw_b, pk_c, pw_c, zeros)

    xb1, Hb2, Hc2, sb2, xc1, sc2 = _tc2_call(
        xb, aggb[0], aggb[1], sb, x_centroid, aggc[0], aggc[1], scn,
        Wb1, bb1, Wc1, bc1)

    aggb2, aggc2 = _get_sc_call()(Hb2, Hc2, pk_b, pw_b, pk_c, pw_c, zeros)

    xbf, xcf = _tc3_call(xb1, aggb2[0], aggb2[1], sb2,
                         xc1, aggc2[0], aggc2[1], sc2)
    return (xbf, xcf)


# speedup vs baseline: 1.9512x; 1.9512x over previous
import functools

import jax
import jax.numpy as jnp
from jax import lax
from jax.experimental import pallas as pl
from jax.experimental.pallas import tpu as pltpu
from jax.experimental.pallas import tpu_sc as plsc

F32 = jnp.float32
HID = 128
NB = 10000
NC = 1000
NN = NB + NC
NUM_LAYERS = 2
K_ED = 40
NW = 32


def _combine(Wsrc, bsrc, Wdst, bdst, Wmlp, bmlp, eps, l):
    hp = jax.lax.Precision.HIGHEST
    i_bb, i_bc, i_cc, i_cb = l * 4 + 0, l * 4 + 1, l * 4 + 2, l * 4 + 3

    def wm(i, j):
        return jnp.dot(Wsrc[i][...] if j is None else j, Wmlp[i][...],
                       preferred_element_type=F32, precision=hp)

    Mb = (1.0 + eps[0, i_bb]) * wm(i_bb, Wdst[i_bb][...]) \
       + (1.0 + eps[0, i_cb]) * wm(i_cb, Wdst[i_cb][...])
    Mc = (1.0 + eps[0, i_bc]) * wm(i_bc, Wdst[i_bc][...]) \
       + (1.0 + eps[0, i_cc]) * wm(i_cc, Wdst[i_cc][...])
    Wb = jnp.concatenate([wm(i_bb, None), wm(i_bc, None), Mb], axis=1)
    Wc = jnp.concatenate([wm(i_cb, None), wm(i_cc, None), Mc], axis=1)

    def bvec(i):
        return bsrc[i][...].reshape(1, HID)

    cb_const = wm(i_bb, bdst[i_bb][...].reshape(1, HID)) + bmlp[i_bb][...].reshape(1, HID) \
             + wm(i_cb, bdst[i_cb][...].reshape(1, HID)) + bmlp[i_cb][...].reshape(1, HID)
    cc_const = wm(i_bc, bdst[i_bc][...].reshape(1, HID)) + bmlp[i_bc][...].reshape(1, HID) \
             + wm(i_cc, bdst[i_cc][...].reshape(1, HID)) + bmlp[i_cc][...].reshape(1, HID)
    bbias = jnp.concatenate([wm(i_bb, bvec(i_bb)), wm(i_bc, bvec(i_bc)), cb_const], axis=1)
    cbias = jnp.concatenate([wm(i_cb, bvec(i_cb)), wm(i_cc, bvec(i_cc)), cc_const], axis=1)
    return Wb, bbias, Wc, cbias


def _split_writes(yb, yc, Hb_o, Hc_o, sb_o, sc_o):
    Hb_o[0:NB, :] = yb[:, 0:HID]
    Hb_o[NB:NN, :] = yc[:, 0:HID]
    Hc_o[0:NB, :] = yb[:, HID:2 * HID]
    Hc_o[NB:NN, :] = yc[:, HID:2 * HID]
    sb_o[...] = yb[:, 2 * HID:]
    sc_o[...] = yc[:, 2 * HID:]


def _tc1_body(old, xc, Wa, ba, Wsrc, bsrc, Wdst, bdst, Wmlp, bmlp, eps,
              xb_o, Hb_o, Hc_o, sb_o, sc_o, Wb1_o, bb1_o, Wc1_o, bc1_o):
    Wb0, bb0, Wc0, bc0 = _combine(Wsrc, bsrc, Wdst, bdst, Wmlp, bmlp, eps, 0)
    Wb1, bb1, Wc1, bc1 = _combine(Wsrc, bsrc, Wdst, bdst, Wmlp, bmlp, eps, 1)
    Wb1_o[...] = Wb1
    bb1_o[...] = bb1
    Wc1_o[...] = Wc1
    bc1_o[...] = bc1
    xb = jnp.dot(old[...], Wa[...], preferred_element_type=F32) + ba[...]
    xb_o[...] = xb
    yb = jnp.dot(xb, Wb0, preferred_element_type=F32) + bb0
    yc = jnp.dot(xc[...], Wc0, preferred_element_type=F32) + bc0
    _split_writes(yb, yc, Hb_o, Hc_o, sb_o, sc_o)


def _tc1_call(old, xc, Wa, ba, Wsrc, bsrc, Wdst, bdst, Wmlp, bmlp, eps):
    b = jax.ShapeDtypeStruct((NB, HID), F32)
    h = jax.ShapeDtypeStruct((NN, HID), F32)
    cs = jax.ShapeDtypeStruct((NC, HID), F32)
    w = jax.ShapeDtypeStruct((HID, 3 * HID), F32)
    bi = jax.ShapeDtypeStruct((1, 3 * HID), F32)
    return pl.pallas_call(
        _tc1_body,
        out_shape=(b, h, h, b, cs, w, bi, w, bi),
    )(old, xc, Wa, ba, Wsrc, bsrc, Wdst, bdst, Wmlp, bmlp, eps)


def _tc2_body(xb, ab0, ab1, sb, xc, ac0, ac1, sc, Wb, bbias, Wc, cbias,
              xb_o, Hb_o, Hc_o, sb_o, xc_o, sc_o):
    xb1 = xb[...] + jax.nn.gelu(ab0[...] + ab1[...] + sb[...])
    xb_o[...] = xb1
    xc1 = xc[...] + jax.nn.gelu(ac0[...] + ac1[...] + sc[...])
    xc_o[...] = xc1
    yb = jnp.dot(xb1, Wb[...], preferred_element_type=F32) + bbias[...]
    yc = jnp.dot(xc1, Wc[...], preferred_element_type=F32) + cbias[...]
    _split_writes(yb, yc, Hb_o, Hc_o, sb_o, sc_o)


def _tc2_call(xb, ab0, ab1, sb, xc, ac0, ac1, sc, Wb, bbias, Wc, cbias):
    b = jax.ShapeDtypeStruct((NB, HID), F32)
    h = jax.ShapeDtypeStruct((NN, HID), F32)
    cs = jax.ShapeDtypeStruct((NC, HID), F32)
    return pl.pallas_call(
        _tc2_body,
        out_shape=(b, h, h, b, cs, cs),
    )(xb, ab0, ab1, sb, xc, ac0, ac1, sc, Wb, bbias, Wc, cbias)


def _tc3_body(xb, ab0, ab1, sb, xc, ac0, ac1, sc, xb_o, xc_o):
    xb_o[...] = xb[...] + jax.nn.gelu(ab0[...] + ab1[...] + sb[...])
    xc_o[...] = xc[...] + jax.nn.gelu(ac0[...] + ac1[...] + sc[...])


def _tc3_call(xb, ab0, ab1, sb, xc, ac0, ac1, sc):
    return pl.pallas_call(
        _tc3_body,
        out_shape=(jax.ShapeDtypeStruct((NB, HID), F32),
                   jax.ShapeDtypeStruct((NC, HID), F32)),
    )(xb, ab0, ab1, sb, xc, ac0, ac1, sc)


def _sc_body(Hb, Hc, s_bb, d_bb, w_bb, s_cb, d_cb, w_cb,
             s_bc, d_bc, w_bc, s_cc, d_cc, w_cc, zeros,
             aggb_o, aggc_o,
             aggb_sh, aggc_sh,
             sidx, didx, wbuf, rows_a, rows_b, sem_a, sem_b):
    c = lax.axis_index("c")
    s = lax.axis_index("s")
    wid = c * 16 + s

    @pl.when(s < 10)
    def _():
        pltpu.sync_copy(zeros.at[:], aggb_sh.at[pl.ds(s * 1000, 1000)])

    @pl.when(s < 5)
    def _():
        pltpu.sync_copy(zeros.at[pl.ds(0, 200)], aggc_sh.at[pl.ds(s * 200, 200)])

    plsc.subcore_barrier()

    def do_rel(htab, src3, dst3, ew3, agg_sh):
        K = K_ED
        iters = src3.shape[1]
        pltpu.sync_copy(src3.at[wid], sidx.at[pl.ds(0, iters)])
        pltpu.sync_copy(dst3.at[wid], didx.at[pl.ds(0, iters)])
        pltpu.sync_copy(ew3.at[wid], wbuf.at[pl.ds(0, iters)])

        def _scale_group(g, w0, lanes, rows):
            wv = wbuf[g, pl.ds(w0, 16)]
            for u in lanes:
                e = w0 + u
                w = wv[u]
                for j in range(8):
                    sl = pl.ds(j * 16, 16)
                    rows[e, sl] = rows[e, sl] * w

        def start(g, rows, sm):
            pltpu.async_copy(htab.at[sidx.at[g]], rows, sm)

        def finish(g, rows, sm):
            pltpu.make_async_copy(htab.at[sidx.at[g]], rows, sm).wait()

            def scale16(t, c2):
                _scale_group(g, t * 16, range(16), rows)
                return c2

            lax.fori_loop(0, K // 16, scale16, 0)
            tail = K % 16
            if tail:
                _scale_group(g, K - 16, range(16 - tail, 16), rows)
            pltpu.sync_copy(rows, agg_sh.at[didx.at[g]], add=True)

        start(0, rows_a, sem_a)

        def body2(t, carry):
            g = t * 2
            start(g + 1, rows_b, sem_b)
            finish(g, rows_a, sem_a)

            @pl.when(g + 2 < iters)
            def _():
                start(g + 2, rows_a, sem_a)

            finish(g + 1, rows_b, sem_b)
            return carry

        lax.fori_loop(0, iters // 2, body2, 0)
        if iters % 2:
            finish(iters - 1, rows_a, sem_a)

    do_rel(Hb, s_bb, d_bb, w_bb, aggb_sh)
    do_rel(Hb, s_cb, d_cb, w_cb, aggb_sh)
    do_rel(Hc, s_bc, d_bc, w_bc, aggc_sh)
    do_rel(Hc, s_cc, d_cc, w_cc, aggc_sh)

    plsc.subcore_barrier()

    @pl.when(s < 10)
    def _():
        pltpu.sync_copy(aggb_sh.at[pl.ds(s * 1000, 1000)],
                        aggb_o.at[c, pl.ds(s * 1000, 1000)])

    @pl.when(s < 5)
    def _():
        pltpu.sync_copy(aggc_sh.at[pl.ds(s * 200, 200)],
                        aggc_o.at[c, pl.ds(s * 200, 200)])


@functools.cache
def _get_sc_call():
  return functools.partial(
    pl.kernel,
    mesh=plsc.VectorSubcoreMesh(core_axis_name="c", subcore_axis_name="s",
                                num_cores=2, num_subcores=16),
    compiler_params=pltpu.CompilerParams(use_tc_tiling_on_sc=False),
    out_type=(jax.ShapeDtypeStruct((2, NB, HID), F32),
              jax.ShapeDtypeStruct((2, NC, HID), F32)),
    scratch_types=[
        pltpu.VMEM_SHARED((NB, HID), F32),
        pltpu.VMEM_SHARED((NC, HID), F32),
        pltpu.VMEM((250, K_ED), jnp.int32),
        pltpu.VMEM((250, K_ED), jnp.int32),
        pltpu.VMEM((250, K_ED), F32),
        pltpu.VMEM((K_ED, HID), F32),
        pltpu.VMEM((K_ED, HID), F32),
        pltpu.SemaphoreType.DMA,
        pltpu.SemaphoreType.DMA,
    ],
  )(_sc_body)


def _pack_edges(src, dst, w):
    per = -(-src.shape[0] // NW)
    iters = -(-per // K_ED)
    pad = NW * iters * K_ED - src.shape[0]

    def p3(x):
        return jnp.pad(x, (0, pad)).reshape(NW, iters, K_ED)

    return p3(src), p3(dst), p3(w)


def kernel(old_data, x_base, x_centroid, edge_index_b2b, edge_index_b2c,
           edge_index_c2c, edge_index_c2b, edge_weight_b2b, edge_weight_b2c,
           edge_weight_c2c, edge_weight_c2b, batch_base, batch_centroid,
           has_edge_attr, params):
    p = params
    ets = [(l, et) for l in range(NUM_LAYERS) for et in ("bb", "bc", "cc", "cb")]
    Wsrc = jnp.stack([p[f"{l}_{et}"]["W_src"] for l, et in ets])
    bsrc = jnp.stack([p[f"{l}_{et}"]["b_src"] for l, et in ets])
    Wdst = jnp.stack([p[f"{l}_{et}"]["W_dst"] for l, et in ets])
    bdst = jnp.stack([p[f"{l}_{et}"]["b_dst"] for l, et in ets])
    Wmlp = jnp.stack([p[f"{l}_{et}"]["W_mlp"] for l, et in ets])
    bmlp = jnp.stack([p[f"{l}_{et}"]["b_mlp"] for l, et in ets])
    eps = jnp.stack([p[f"{l}_{et}"]["eps"] for l, et in ets]).reshape(1, 8)

    e_bb = _pack_edges(edge_index_b2b[0], edge_index_b2b[1], edge_weight_b2b)
    e_cb = _pack_edges(edge_index_c2b[0] + NB, edge_index_c2b[1], edge_weight_c2b)
    e_bc = _pack_edges(edge_index_b2c[0], edge_index_b2c[1], edge_weight_b2c)
    e_cc = _pack_edges(edge_index_c2c[0] + NB, edge_index_c2c[1], edge_weight_c2c)

    zeros = jnp.zeros((1000, HID), F32)

    (xb, Hb, Hc, sb, scn, Wb1, bb1, Wc1, bc1) = _tc1_call(
        old_data, x_centroid, p["W_atom"], p["b_atom"].reshape(1, HID),
        Wsrc, bsrc, Wdst, bdst, Wmlp, bmlp, eps)

    aggb, aggc = _get_sc_call()(Hb, Hc, *e_bb, *e_cb, *e_bc, *e_cc, zeros)

    xb1, Hb2, Hc2, sb2, xc1, sc2 = _tc2_call(
        xb, aggb[0], aggb[1], sb, x_centroid, aggc[0], aggc[1], scn,
        Wb1, bb1, Wc1, bc1)

    aggb2, aggc2 = _get_sc_call()(Hb2, Hc2, *e_bb, *e_cb, *e_bc, *e_cc, zeros)

    xbf, xcf = _tc3_call(xb1, aggb2[0], aggb2[1], sb2,
                         xc1, aggc2[0], aggc2[1], sc2)
    return (xbf, xcf)
